# Initial kernel scaffold; baseline (speedup 1.0000x reference)
import jax
import jax.numpy as jnp
from jax.experimental import pallas as pl


def kernel(indices, tok_table, pos_table):
    raise NotImplementedError("write your pallas kernel here")


# speedup vs baseline: 4.3574x; 4.3574x over previous
import functools

import jax
import jax.numpy as jnp
from jax import lax
from jax.experimental import pallas as pl
from jax.experimental.pallas import tpu as pltpu
from jax.experimental.pallas import tpu_sc as plsc

VOCAB = 100000
TOK_DIM = 32
MAX_LEN = 200
POS_DIM = 32
BATCH = 4096

OUT_DIM = TOK_DIM + POS_DIM
N_TOK = BATCH * MAX_LEN
NC, NS = 2, 16
NW = NC * NS
TOK_PER_W = N_TOK // NW
IDX_W = 128
K = 8
CHUNK = K * IDX_W
ITERS = TOK_PER_W // CHUNK
B_PER_W = BATCH // NW
PREP = 8
P_ITERS = B_PER_W // PREP


@functools.partial(
    pl.kernel,
    mesh=plsc.VectorSubcoreMesh(core_axis_name="c", subcore_axis_name="s"),
    out_type=jax.ShapeDtypeStruct((N_TOK, OUT_DIM), jnp.float32),
    compiler_params=pltpu.CompilerParams(use_tc_tiling_on_sc=False),
    scratch_types=[
        pltpu.VMEM((K, IDX_W), jnp.int32),
        pltpu.VMEM((CHUNK, TOK_DIM), jnp.float32),
        pltpu.VMEM((PREP * MAX_LEN, POS_DIM), jnp.float32),
        pltpu.SemaphoreType.DMA,
        pltpu.SemaphoreType.DMA,
    ],
)
def _emb(tok_hbm, idx_hbm, pos_hbm, out_hbm, idx_v, rows_v, pos_v, sem, psem):
    wid = lax.axis_index("s") * NC + lax.axis_index("c")
    tok0 = wid * TOK_PER_W
    row0 = wid * (TOK_PER_W // IDX_W)

    for r in range(PREP):
        pltpu.sync_copy(pos_hbm, pos_v.at[pl.ds(r * MAX_LEN, MAX_LEN)])

    def body(it, carry):
        tbase = tok0 + it * CHUNK
        rbase = row0 + it * K
        pltpu.sync_copy(idx_hbm.at[pl.ds(rbase, K)], idx_v)
        gathers = [
            pltpu.async_copy(
                tok_hbm.at[idx_v.at[j]],
                rows_v.at[pl.ds(j * IDX_W, IDX_W)],
                sem,
            )
            for j in range(K)
        ]
        for g in gathers:
            g.wait()
        pltpu.sync_copy(
            rows_v, out_hbm.at[pl.ds(tbase, CHUNK), pl.ds(0, TOK_DIM)]
        )
        return carry

    lax.fori_loop(0, ITERS, body, 0)

    def pbody(i, carry):
        pbase = tok0 + i * (PREP * MAX_LEN)
        pltpu.sync_copy(
            pos_v,
            out_hbm.at[pl.ds(pbase, PREP * MAX_LEN), pl.ds(TOK_DIM, POS_DIM)],
        )
        return carry

    lax.fori_loop(0, P_ITERS, pbody, 0)


def kernel(indices, tok_table, pos_table):
    idx2 = indices.reshape(-1, IDX_W).astype(jnp.int32)
    out = _emb(tok_table, idx2, pos_table)
    return out.reshape(BATCH, MAX_LEN, OUT_DIM)


# speedup vs baseline: 4.4070x; 1.0114x over previous
import functools

import jax
import jax.numpy as jnp
from jax import lax
from jax.experimental import pallas as pl
from jax.experimental.pallas import tpu as pltpu
from jax.experimental.pallas import tpu_sc as plsc

VOCAB = 100000
TOK_DIM = 32
MAX_LEN = 200
POS_DIM = 32
BATCH = 4096

OUT_DIM = TOK_DIM + POS_DIM
N_TOK = BATCH * MAX_LEN
NC, NS = 2, 16
NW = NC * NS
TOK_PER_W = N_TOK // NW
CHUNK = 1280
ITERS = TOK_PER_W // CHUNK
B_PER_W = BATCH // NW
PREP = 8
P_ITERS = B_PER_W // PREP


@functools.partial(
    pl.kernel,
    mesh=plsc.VectorSubcoreMesh(core_axis_name="c", subcore_axis_name="s"),
    out_type=jax.ShapeDtypeStruct((N_TOK, OUT_DIM), jnp.float32),
    compiler_params=pltpu.CompilerParams(use_tc_tiling_on_sc=False),
    scratch_types=[
        pltpu.VMEM((CHUNK,), jnp.int32),
        pltpu.VMEM((CHUNK, TOK_DIM), jnp.float32),
        pltpu.VMEM((PREP * MAX_LEN, POS_DIM), jnp.float32),
        pltpu.SemaphoreType.DMA,
        pltpu.SemaphoreType.DMA,
    ],
)
def _emb(tok_hbm, idx_hbm, pos_hbm, out_hbm, idx_v, rows_v, pos_v, sem, psem):
    wid = lax.axis_index("s") * NC + lax.axis_index("c")
    tok0 = wid * TOK_PER_W

    for r in range(PREP):
        pltpu.sync_copy(pos_hbm, pos_v.at[pl.ds(r * MAX_LEN, MAX_LEN)])

    def body(it, carry):
        tbase = tok0 + it * CHUNK
        pltpu.sync_copy(idx_hbm.at[pl.ds(tbase, CHUNK)], idx_v)
        pltpu.async_copy(tok_hbm.at[idx_v], rows_v, sem).wait()
        pltpu.sync_copy(
            rows_v, out_hbm.at[pl.ds(tbase, CHUNK), pl.ds(0, TOK_DIM)]
        )
        return carry

    lax.fori_loop(0, ITERS, body, 0)

    def pbody(i, carry):
        pbase = tok0 + i * (PREP * MAX_LEN)
        pltpu.sync_copy(
            pos_v,
            out_hbm.at[pl.ds(pbase, PREP * MAX_LEN), pl.ds(TOK_DIM, POS_DIM)],
        )
        return carry

    lax.fori_loop(0, P_ITERS, pbody, 0)


def kernel(indices, tok_table, pos_table):
    idx_flat = indices.reshape(-1).astype(jnp.int32)
    out = _emb(tok_table, idx_flat, pos_table)
    return out.reshape(BATCH, MAX_LEN, OUT_DIM)


# speedup vs baseline: 4.4972x; 1.0205x over previous
import functools

import jax
import jax.numpy as jnp
from jax import lax
from jax.experimental import pallas as pl
from jax.experimental.pallas import tpu as pltpu
from jax.experimental.pallas import tpu_sc as plsc

VOCAB = 100000
TOK_DIM = 32
MAX_LEN = 200
POS_DIM = 32
BATCH = 4096

OUT_DIM = TOK_DIM + POS_DIM
N_TOK = BATCH * MAX_LEN
NC, NS = 2, 16
NW = NC * NS
TOK_PER_W = N_TOK // NW
CHUNK = 1024
ITERS = TOK_PER_W // CHUNK
B_PER_W = BATCH // NW
PREP = 8
P_ITERS = B_PER_W // PREP
NBUF = 2


@functools.partial(
    pl.kernel,
    mesh=plsc.VectorSubcoreMesh(core_axis_name="c", subcore_axis_name="s"),
    out_type=jax.ShapeDtypeStruct((N_TOK, OUT_DIM), jnp.float32),
    compiler_params=pltpu.CompilerParams(use_tc_tiling_on_sc=False),
    scratch_types=[
        pltpu.VMEM((NBUF * CHUNK,), jnp.int32),
        pltpu.VMEM((NBUF * CHUNK, TOK_DIM), jnp.float32),
        pltpu.VMEM((PREP * MAX_LEN, POS_DIM), jnp.float32),
        pltpu.SemaphoreType.DMA,
        pltpu.SemaphoreType.DMA,
        pltpu.SemaphoreType.DMA,
        pltpu.SemaphoreType.DMA,
    ],
)
def _emb(tok_hbm, idx_hbm, pos_hbm, out_hbm, idx_v, rows_v, pos_v,
         isem, gsem, ssem, psem):
    wid = lax.axis_index("s") * NC + lax.axis_index("c")
    tok0 = wid * TOK_PER_W

    def idx_load(g, fire):
        buf = lax.rem(g, NBUF) * CHUNK
        d = pltpu.make_async_copy(
            idx_hbm.at[pl.ds(tok0 + g * CHUNK, CHUNK)],
            idx_v.at[pl.ds(buf, CHUNK)], isem)
        d.start() if fire else d.wait()

    def gather(g, fire):
        buf = lax.rem(g, NBUF) * CHUNK
        d = pltpu.make_async_copy(
            tok_hbm.at[idx_v.at[pl.ds(buf, CHUNK)]],
            rows_v.at[pl.ds(buf, CHUNK)], gsem)
        d.start() if fire else d.wait()

    def store(g, fire):
        buf = lax.rem(g, NBUF) * CHUNK
        d = pltpu.make_async_copy(
            rows_v.at[pl.ds(buf, CHUNK)],
            out_hbm.at[pl.ds(tok0 + g * CHUNK, CHUNK), pl.ds(0, TOK_DIM)],
            ssem)
        d.start() if fire else d.wait()

    def pos_store(i, fire):
        d = pltpu.make_async_copy(
            pos_v,
            out_hbm.at[pl.ds(tok0 + i * (PREP * MAX_LEN), PREP * MAX_LEN),
                       pl.ds(TOK_DIM, POS_DIM)], psem)
        d.start() if fire else d.wait()

    for r in range(PREP):
        pltpu.sync_copy(pos_hbm, pos_v.at[pl.ds(r * MAX_LEN, MAX_LEN)])

    idx_load(0, True)
    idx_load(1, True)
    idx_load(0, False)
    gather(0, True)

    def main(g, carry):
        gather(g, False)
        store(g, True)
        pl.when(g < ITERS - 2)(lambda: idx_load(g + 2, True))
        idx_load(g + 1, False)
        pl.when(g >= 1)(lambda: store(g - 1, False))
        gather(g + 1, True)
        pl.when(g < P_ITERS)(lambda: pos_store(g, True))
        return carry

    lax.fori_loop(0, ITERS - 1, main, 0)

    gather(ITERS - 1, False)
    store(ITERS - 1, True)
    store(ITERS - 2, False)
    store(ITERS - 1, False)

    def pdrain(i, carry):
        pos_store(i, False)
        return carry

    lax.fori_loop(0, P_ITERS, pdrain, 0)


def kernel(indices, tok_table, pos_table):
    idx_flat = indices.reshape(-1).astype(jnp.int32)
    out = _emb(tok_table, idx_flat, pos_table)
    return out.reshape(BATCH, MAX_LEN, OUT_DIM)
